# Initial kernel scaffold; baseline (speedup 1.0000x reference)
import jax
import jax.numpy as jnp
from jax.experimental import pallas as pl


def kernel(features, mask, codebook, count, avg):
    raise NotImplementedError("write your pallas kernel here")


# speedup vs baseline: 3.4107x; 3.4107x over previous
import jax
import jax.numpy as jnp
from jax.experimental import pallas as pl
from jax.experimental.pallas import tpu as pltpu

_N = 2048
_K = 1024
_D = 32
_BN = 256
_NB = _N // _BN
_GAMMA = 0.99


def _vq_tc_kernel(f_ref, ct_ref, cb_ref, maskf_ref, f_full_ref, maskf_full_ref,
                  cnt_ref, avg_ref,
                  outf_ref, af_ref, loss_ref, unas_ref,
                  cnt_out_ref, avg_out_ref, cb_out_ref,
                  colmin_ref, colarg_ref, hist_ref, fsum_ref, loss_acc_ref):
    b = pl.program_id(0)

    @pl.when(b == 0)
    def _init():
        colmin_ref[:] = jnp.full((1, _K), jnp.inf, jnp.float32)
        colarg_ref[:] = jnp.zeros((1, _K), jnp.int32)
        hist_ref[:] = jnp.zeros((_K, 1), jnp.float32)
        fsum_ref[:] = jnp.zeros((_K, _D), jnp.float32)
        loss_acc_ref[:] = jnp.zeros((1, 1), jnp.float32)

    f = f_ref[:]
    ct = ct_ref[:]
    cb = cb_ref[:]
    maskf = maskf_ref[:]

    acc = jnp.zeros((_BN, _K), jnp.float32)
    for d in range(_D):
        diff = f[:, d:d + 1] - ct[d:d + 1, :]
        acc = acc + diff * diff

    inf = jnp.float32(jnp.inf)
    dm = jnp.where(maskf > 0, acc, inf)

    lane_iota = jax.lax.broadcasted_iota(jnp.int32, (_BN, _K), 1)
    sub_iota = jax.lax.broadcasted_iota(jnp.int32, (_BN, _K), 0)

    min_row = jnp.min(dm, axis=1, keepdims=True)
    af = jnp.min(jnp.where(dm == min_row, lane_iota, _K),
                 axis=1, keepdims=True).astype(jnp.int32)

    blk_colmin = jnp.min(dm, axis=0, keepdims=True)
    blk_colarg = jnp.min(jnp.where(dm == blk_colmin, sub_iota + b * _BN, _N),
                         axis=0, keepdims=True).astype(jnp.int32)
    better = blk_colmin < colmin_ref[:]
    colarg_ref[:] = jnp.where(better, blk_colarg, colarg_ref[:])
    colmin_ref[:] = jnp.minimum(blk_colmin, colmin_ref[:])

    onehot = (lane_iota == af).astype(jnp.float32)
    outf_ref[:] = jnp.dot(onehot, cb, preferred_element_type=jnp.float32)
    af_ref[:] = af
    hist_ref[:] = hist_ref[:] + jax.lax.dot_general(
        onehot, maskf, (((0,), (0,)), ((), ())),
        preferred_element_type=jnp.float32)
    fsum_ref[:] = fsum_ref[:] + jax.lax.dot_general(
        onehot, f, (((0,), (0,)), ((), ())),
        preferred_element_type=jnp.float32)
    loss_acc_ref[:] = loss_acc_ref[:] + jnp.sum(
        jnp.where(maskf > 0, min_row, 0.0) / _D, keepdims=True)

    @pl.when(b == _NB - 1)
    def _finalize():
        f_full = f_full_ref[:]
        maskf_full = maskf_full_ref[:]
        ar = colarg_ref[:]
        sub_iota_full = jax.lax.broadcasted_iota(jnp.int32, (_N, _K), 0)
        onehot_rev = (sub_iota_full == ar).astype(jnp.float32)
        frev = jax.lax.dot_general(onehot_rev, f_full, (((0,), (0,)), ((), ())),
                                   preferred_element_type=jnp.float32)
        hist = hist_ref[:]
        fsum = fsum_ref[:]
        total = jnp.maximum(jnp.sum(maskf_full), 1.0)
        loss_ref[:] = loss_acc_ref[:] / total

        g = _GAMMA
        cnt_new = (1 - g) * hist + g * cnt_ref[:]
        avg_new = (1 - g) * hist / total + g * avg_ref[:]
        alpha = jnp.exp(-avg_new * _K * 10 / (1 - g) - 0.001)
        assigned = (g * cb + (1 - g) * fsum) / jnp.maximum(cnt_new, 1.0)
        unassigned = (1 - alpha) * cb + alpha * frev
        upd = jnp.where(hist < 1, assigned, unassigned)
        cb_out_ref[:] = cb + (cb - upd)
        cnt_out_ref[:] = cnt_new
        avg_out_ref[:] = avg_new
        unas_ref[:] = jnp.sum((hist > 0).astype(jnp.float32),
                              keepdims=True) / _K


def kernel(features, mask, codebook, count, avg):
    maskf = mask.astype(jnp.float32).reshape(_N, 1)
    ct = codebook.T
    cnt = count.reshape(_K, 1)
    av = avg.reshape(_K, 1)
    full = lambda s: pl.BlockSpec(s, lambda b: (0, 0))
    outs = pl.pallas_call(
        _vq_tc_kernel,
        grid=(_NB,),
        in_specs=[
            pl.BlockSpec((_BN, _D), lambda b: (b, 0)),
            full((_D, _K)),
            full((_K, _D)),
            pl.BlockSpec((_BN, 1), lambda b: (b, 0)),
            full((_N, _D)),
            full((_N, 1)),
            full((_K, 1)),
            full((_K, 1)),
        ],
        out_specs=[
            pl.BlockSpec((_BN, _D), lambda b: (b, 0)),
            pl.BlockSpec((_BN, 1), lambda b: (b, 0)),
            full((1, 1)),
            full((1, 1)),
            full((_K, 1)),
            full((_K, 1)),
            full((_K, _D)),
        ],
        out_shape=[
            jax.ShapeDtypeStruct((_N, _D), jnp.float32),
            jax.ShapeDtypeStruct((_N, 1), jnp.int32),
            jax.ShapeDtypeStruct((1, 1), jnp.float32),
            jax.ShapeDtypeStruct((1, 1), jnp.float32),
            jax.ShapeDtypeStruct((_K, 1), jnp.float32),
            jax.ShapeDtypeStruct((_K, 1), jnp.float32),
            jax.ShapeDtypeStruct((_K, _D), jnp.float32),
        ],
        scratch_shapes=[
            pltpu.VMEM((1, _K), jnp.float32),
            pltpu.VMEM((1, _K), jnp.int32),
            pltpu.VMEM((_K, 1), jnp.float32),
            pltpu.VMEM((_K, _D), jnp.float32),
            pltpu.VMEM((1, 1), jnp.float32),
        ],
    )(features, ct, codebook, maskf, features, maskf, cnt, av)
    outf, af, loss, unas, cnt_o, avg_o, cb_o = outs
    return (outf, af.reshape(_N), loss.reshape(()), unas.reshape(()),
            cnt_o.reshape(_K), avg_o.reshape(_K), cb_o)
